# baseline (speedup 1.0000x reference)
import functools

import jax
import jax.numpy as jnp
import numpy as np
from jax import lax
from jax.experimental import pallas as pl
from jax.experimental.pallas import tpu as pltpu
from jax.experimental.pallas import tpu_sc as plsc

N, E, DIN, DEDGE, H, R, G, DEPTH = 10000, 320000, 128, 16, 300, 1024, 64, 5
HP = 320
NQ = 4
QW = HP // NQ
NC, NS = 2, 16
EPT = E // NS
EB = 40
NBLK = EPT // EB
EBLK = E // EB
NCHUNK = N // EB
KCH = (NCHUNK + NS - 1) // NS
NROWB = 25
RB = N // NROWB
EROWB = 50
ERB = E // EROWB
NSLOT = 5
VL = 16
EW = 128


def _proj_nodes_body(x_ref, wn_ref, bn_ref, out_ref):
    h = jnp.dot(x_ref[...], wn_ref[...], preferred_element_type=jnp.float32)
    h = jnp.maximum(h + bn_ref[...], 0.0)
    for q in range(NQ):
        out_ref[q] = h[:, q * QW:(q + 1) * QW]


def _proj_edges_body(ea_ref, we_ref, be_ref, out_ref):
    e = jnp.dot(ea_ref[...], we_ref[0], preferred_element_type=jnp.float32)
    e = e + be_ref[0]
    out_ref[...] = jnp.concatenate(
        [e, jnp.zeros((ERB, EW - QW), jnp.float32)], axis=1)


def _mlp_body(h_ref, a_ref, w1_ref, b1_ref, w2_ref, b2_ref,
              out_ref, *, last):
    hb = jnp.concatenate([h_ref[q] for q in range(NQ)], axis=1)
    ab = jnp.concatenate([a_ref[q] for q in range(NQ)], axis=1)
    hb = hb + ab
    t = jnp.dot(hb, w1_ref[...], preferred_element_type=jnp.float32)
    t = jnp.maximum(t + b1_ref[...], 0.0)
    o = jnp.dot(t, w2_ref[...], preferred_element_type=jnp.float32)
    o = o + b2_ref[...]
    if not last:
        o = jnp.maximum(o, 0.0)
    for q in range(NQ):
        out_ref[q] = o[:, q * QW:(q + 1) * QW]


def _readout_body(h_ref, batch_ref, ws_ref, bs_ref, pw_ref, out_ref, acc_ref):
    i = pl.program_id(0)

    @pl.when(i == 0)
    def _():
        acc_ref[...] = jnp.zeros_like(acc_ref)

    hb = jnp.concatenate([h_ref[q] for q in range(NQ)], axis=1)
    bv = batch_ref[0, 0, :]
    gid = lax.broadcasted_iota(jnp.int32, (RB, G), 1)
    onehot = (bv[:, None] == gid).astype(jnp.float32)
    acc_ref[...] += lax.dot_general(
        onehot, hb, (((0,), (0,)), ((), ())),
        preferred_element_type=jnp.float32)

    @pl.when(i == NROWB - 1)
    def _():
        ro = jnp.dot(acc_ref[...], ws_ref[...],
                     preferred_element_type=jnp.float32) + bs_ref[...]
        pw = pw_ref[0, 0]
        out_ref[...] = jnp.where(ro >= 0, ro, pw * ro)


def _mp_body(h_ref, e_ref, srcb_ref, dstb_ref, out_ref,
             sv_all, dv_all, mv, ev, agg_sh, gsem, ssem):
    c = lax.axis_index("c")
    s = lax.axis_index("s")

    pltpu.sync_copy(dstb_ref.at[pl.ds(s * EPT, EPT)], dv_all)

    for ph in range(2):
        _mp_phase(2 * ph + c, c, s, h_ref, e_ref, srcb_ref, out_ref,
                  sv_all, dv_all, mv, ev, agg_sh, gsem, ssem)


def _mp_phase(q, c, s, h_ref, e_ref, srcb_ref, out_ref,
              sv_all, dv_all, mv, ev, agg_sh, gsem, ssem):
    pltpu.sync_copy(srcb_ref.at[pl.ds(q * E + s * EPT, EPT)], sv_all)

    def zrow(r, carry):
        for j in range(QW // VL):
            mv[0][r, pl.ds(j * VL, VL)] = jnp.zeros((VL,), jnp.float32)
        return carry

    lax.fori_loop(0, EB, zrow, 0)

    def zchunk(k, carry):
        chunk = s + k * NS

        @pl.when(chunk < NCHUNK)
        def _():
            pltpu.sync_copy(mv[0], agg_sh.at[pl.ds(chunk * EB, EB)])

        return carry

    lax.fori_loop(0, KCH, zchunk, 0)
    plsc.subcore_barrier()

    ebase = q * E + s * EPT

    def issue_main(b, k):
        pltpu.async_copy(
            e_ref.at[pl.ds(ebase + b * EB, EB), pl.ds(0, QW)], ev[k],
            gsem[k])
        pltpu.async_copy(h_ref.at[sv_all.at[pl.ds(b * EB, EB)]], mv[k],
                         gsem[k])

    def wait_main(k):
        pltpu.make_async_copy(e_ref.at[pl.ds(ebase, EB), pl.ds(0, QW)],
                              ev[k], gsem[k]).wait()
        pltpu.make_async_copy(h_ref.at[sv_all.at[pl.ds(0, EB)]], mv[k],
                              gsem[k]).wait()

    def issue_scatter(b, k):
        pltpu.async_copy(mv[k], agg_sh.at[dv_all.at[pl.ds(b * EB, EB)]],
                         ssem[k], add=True)

    def wait_scatter(k):
        pltpu.make_async_copy(mv[k], agg_sh.at[dv_all.at[pl.ds(0, EB)]],
                              ssem[k]).wait()

    def compute(k):
        mvk, evk = mv[k], ev[k]

        def row4(r4, carry):
            for dr in range(4):
                r = r4 * 4 + dr
                for j in range(QW // VL):
                    sl = pl.ds(j * VL, VL)
                    mvk[r, sl] = jnp.maximum(mvk[r, sl] + evk[r, sl], 0.0)
            return carry

        lax.fori_loop(0, EB // 4, row4, 0)

    issue_main(0, 0)
    issue_main(1, 1)
    issue_main(2, 2)

    def outer(t, carry):
        for k in range(NSLOT):
            b = NSLOT * t + k
            wait_main(k)
            compute(k)
            issue_scatter(b, k)
            k2 = (k + 3) % NSLOT
            if k < 2:
                @pl.when(t > 0)
                def _():
                    wait_scatter(k2)

                issue_main(b + 3, k2)
            else:
                wait_scatter(k2)

                @pl.when(t < NBLK // NSLOT - 1)
                def _():
                    issue_main(b + 3, k2)
        return carry

    lax.fori_loop(0, NBLK // NSLOT, outer, 0)
    wait_scatter(3)
    wait_scatter(4)
    plsc.subcore_barrier()

    def ochunk(k, carry):
        chunk = s + k * NS

        @pl.when(chunk < NCHUNK)
        def _():
            pltpu.sync_copy(agg_sh.at[pl.ds(chunk * EB, EB)],
                            out_ref.at[pl.ds(q * N + chunk * EB, EB)])

        return carry

    lax.fori_loop(0, KCH, ochunk, 0)
    plsc.subcore_barrier()


def _make_mp():
    return pl.kernel(
        _mp_body,
        out_type=jax.ShapeDtypeStruct((NQ * N, QW), jnp.float32),
        mesh=plsc.VectorSubcoreMesh(core_axis_name="c", subcore_axis_name="s",
                                    num_cores=NC, num_subcores=NS),
        scratch_types=[
            pltpu.VMEM((EPT,), jnp.int32),
            pltpu.VMEM((EPT,), jnp.int32),
            [pltpu.VMEM((EB, QW), jnp.float32) for _ in range(NSLOT)],
            [pltpu.VMEM((EB, QW), jnp.float32) for _ in range(NSLOT)],
            pltpu.VMEM_SHARED((N, QW), jnp.float32),
            [pltpu.SemaphoreType.DMA for _ in range(NSLOT)],
            [pltpu.SemaphoreType.DMA for _ in range(NSLOT)],
        ],
        compiler_params=pltpu.CompilerParams(use_tc_tiling_on_sc=False),
    )


_mp = _make_mp()


def kernel(x, edge_attr, edge_index, batch, Wn, bn, We, be, W1, b1, W2, b2,
           Ws, bs, pw):
    f32 = jnp.float32
    src = edge_index[0]
    dst = edge_index[1]
    qoff = (jnp.arange(NQ, dtype=jnp.int32) * N)[:, None]
    src4 = (src[None, :] + qoff).reshape(NQ * E)
    pad = HP - H
    Wn_p = jnp.pad(Wn, ((0, 0), (0, pad)))
    bn_p = jnp.pad(bn, (0, pad)).reshape(1, HP)
    We_p = jnp.pad(We, ((0, 0), (0, pad)))
    be_p = jnp.pad(be, (0, pad)).reshape(1, HP)
    W1_p = jnp.pad(W1, ((0, 0), (0, pad), (0, pad)))
    b1_p = jnp.pad(b1, ((0, 0), (0, pad)))
    W2_p = jnp.pad(W2, ((0, 0), (0, pad), (0, pad)))
    b2_p = jnp.pad(b2, ((0, 0), (0, pad)))
    Ws_p = jnp.pad(Ws, ((0, pad), (0, 0)))

    h = pl.pallas_call(
        _proj_nodes_body,
        grid=(NROWB,),
        in_specs=[
            pl.BlockSpec((RB, DIN), lambda i: (i, 0)),
            pl.BlockSpec((DIN, HP), lambda i: (0, 0)),
            pl.BlockSpec((1, HP), lambda i: (0, 0)),
        ],
        out_specs=pl.BlockSpec((NQ, RB, QW), lambda i: (0, i, 0)),
        out_shape=jax.ShapeDtypeStruct((NQ, N, QW), f32),
    )(x, Wn_p, bn_p)

    e_flat = pl.pallas_call(
        _proj_edges_body,
        grid=(NQ, EROWB),
        in_specs=[
            pl.BlockSpec((ERB, DEDGE), lambda q, i: (i, 0)),
            pl.BlockSpec((1, DEDGE, QW), lambda q, i: (q, 0, 0)),
            pl.BlockSpec((1, 1, QW), lambda q, i: (q, 0, 0)),
        ],
        out_specs=pl.BlockSpec((ERB, EW), lambda q, i: (q * EROWB + i, 0)),
        out_shape=jax.ShapeDtypeStruct((NQ * E, EW), f32),
    )(edge_attr,
      We_p.reshape(DEDGE, NQ, QW).transpose(1, 0, 2),
      be_p.reshape(1, NQ, QW).transpose(1, 0, 2))

    for i in range(DEPTH):
        h_flat = h.reshape(NQ * N, QW)
        agg = _mp(h_flat, e_flat, src4, dst)
        h = pl.pallas_call(
            functools.partial(_mlp_body, last=(i == DEPTH - 1)),
            grid=(NROWB,),
            in_specs=[
                pl.BlockSpec((NQ, RB, QW), lambda i: (0, i, 0)),
                pl.BlockSpec((NQ, RB, QW), lambda i: (0, i, 0)),
                pl.BlockSpec((HP, HP), lambda i: (0, 0)),
                pl.BlockSpec((1, HP), lambda i: (0, 0)),
                pl.BlockSpec((HP, HP), lambda i: (0, 0)),
                pl.BlockSpec((1, HP), lambda i: (0, 0)),
            ],
            out_specs=pl.BlockSpec((NQ, RB, QW), lambda i: (0, i, 0)),
            out_shape=jax.ShapeDtypeStruct((NQ, N, QW), f32),
        )(h, agg.reshape(NQ, N, QW),
          W1_p[i], b1_p[i].reshape(1, HP), W2_p[i], b2_p[i].reshape(1, HP))

    out = pl.pallas_call(
        _readout_body,
        grid=(NROWB,),
        in_specs=[
            pl.BlockSpec((NQ, RB, QW), lambda i: (0, i, 0)),
            pl.BlockSpec((1, 1, RB), lambda i: (i, 0, 0)),
            pl.BlockSpec((HP, R), lambda i: (0, 0)),
            pl.BlockSpec((1, R), lambda i: (0, 0)),
            pl.BlockSpec((1, 1), lambda i: (0, 0)),
        ],
        out_specs=pl.BlockSpec((G, R), lambda i: (0, 0)),
        out_shape=jax.ShapeDtypeStruct((G, R), f32),
        scratch_shapes=[pltpu.VMEM((G, HP), f32)],
    )(h, batch.reshape(NROWB, 1, RB), Ws_p, bs.reshape(1, R),
      pw.reshape(1, 1))
    return out


# speedup vs baseline: 1.0229x; 1.0229x over previous
import functools

import jax
import jax.numpy as jnp
import numpy as np
from jax import lax
from jax.experimental import pallas as pl
from jax.experimental.pallas import tpu as pltpu
from jax.experimental.pallas import tpu_sc as plsc

N, E, DIN, DEDGE, H, R, G, DEPTH = 10000, 320000, 128, 16, 300, 1024, 64, 5
HP = 320
NQ = 4
QW = HP // NQ
NC, NS = 2, 16
EPT = E // NS
EB = 40
NBLK = EPT // EB
EBLK = E // EB
NCHUNK = N // EB
KCH = (NCHUNK + NS - 1) // NS
NROWB = 25
RB = N // NROWB
EROWB = 50
ERB = E // EROWB
NSLOT = 5
VL = 16
EW = 128


def _proj_nodes_body(x_ref, wn_ref, bn_ref, out_ref):
    h = jnp.dot(x_ref[...], wn_ref[...], preferred_element_type=jnp.float32)
    h = jnp.maximum(h + bn_ref[...], 0.0)
    for q in range(NQ):
        out_ref[q] = h[:, q * QW:(q + 1) * QW]


def _proj_edges_body(ea_ref, we_ref, be_ref, out_ref):
    e = jnp.dot(ea_ref[...], we_ref[0], preferred_element_type=jnp.float32)
    e = e + be_ref[0]
    out_ref[...] = jnp.concatenate(
        [e, jnp.zeros((ERB, EW - QW), jnp.float32)], axis=1)


def _mlp_body(h_ref, a_ref, w1_ref, b1_ref, w2_ref, b2_ref,
              out_ref, *, last):
    hb = jnp.concatenate([h_ref[q] for q in range(NQ)], axis=1)
    ab = jnp.concatenate([a_ref[q] for q in range(NQ)], axis=1)
    hb = hb + ab
    t = jnp.dot(hb, w1_ref[...], preferred_element_type=jnp.float32)
    t = jnp.maximum(t + b1_ref[...], 0.0)
    o = jnp.dot(t, w2_ref[...], preferred_element_type=jnp.float32)
    o = o + b2_ref[...]
    if not last:
        o = jnp.maximum(o, 0.0)
    for q in range(NQ):
        out_ref[q] = o[:, q * QW:(q + 1) * QW]


def _readout_body(h_ref, batch_ref, ws_ref, bs_ref, pw_ref, out_ref, acc_ref):
    i = pl.program_id(0)

    @pl.when(i == 0)
    def _():
        acc_ref[...] = jnp.zeros_like(acc_ref)

    hb = jnp.concatenate([h_ref[q] for q in range(NQ)], axis=1)
    bv = batch_ref[0, 0, :]
    gid = lax.broadcasted_iota(jnp.int32, (RB, G), 1)
    onehot = (bv[:, None] == gid).astype(jnp.float32)
    acc_ref[...] += lax.dot_general(
        onehot, hb, (((0,), (0,)), ((), ())),
        preferred_element_type=jnp.float32)

    @pl.when(i == NROWB - 1)
    def _():
        ro = jnp.dot(acc_ref[...], ws_ref[...],
                     preferred_element_type=jnp.float32) + bs_ref[...]
        pw = pw_ref[0, 0]
        out_ref[...] = jnp.where(ro >= 0, ro, pw * ro)


def _mp_body(h_ref, e_ref, srcb_ref, dstb_ref, out_ref,
             sv_all, dv_all, mv, ev, agg_sh, gsem, ssem):
    c = lax.axis_index("c")
    s = lax.axis_index("s")

    pltpu.sync_copy(dstb_ref.at[pl.ds(s * EPT, EPT)], dv_all)

    for ph in range(2):
        _mp_phase(2 * ph + c, c, s, h_ref, e_ref, srcb_ref, out_ref,
                  sv_all, dv_all, mv, ev, agg_sh, gsem, ssem)


def _mp_phase(q, c, s, h_ref, e_ref, srcb_ref, out_ref,
              sv_all, dv_all, mv, ev, agg_sh, gsem, ssem):
    pltpu.sync_copy(srcb_ref.at[pl.ds(q * E + s * EPT, EPT)], sv_all)

    def zrow(r, carry):
        for j in range(QW // VL):
            mv[0][r, pl.ds(j * VL, VL)] = jnp.zeros((VL,), jnp.float32)
        return carry

    lax.fori_loop(0, EB, zrow, 0)

    def zchunk(k, carry):
        chunk = s + k * NS

        @pl.when(chunk < NCHUNK)
        def _():
            pltpu.async_copy(mv[0], agg_sh.at[pl.ds(chunk * EB, EB)],
                             ssem[0])

        return carry

    def zwait(k, carry):
        chunk = s + k * NS

        @pl.when(chunk < NCHUNK)
        def _():
            pltpu.make_async_copy(mv[0], agg_sh.at[pl.ds(0, EB)],
                                  ssem[0]).wait()

        return carry

    lax.fori_loop(0, KCH, zchunk, 0)
    lax.fori_loop(0, KCH, zwait, 0)
    plsc.subcore_barrier()

    ebase = q * E + s * EPT

    def issue_main(b, k):
        pltpu.async_copy(
            e_ref.at[pl.ds(ebase + b * EB, EB), pl.ds(0, QW)], ev[k],
            gsem[k])
        pltpu.async_copy(h_ref.at[sv_all.at[pl.ds(b * EB, EB)]], mv[k],
                         gsem[k])

    def wait_main(k):
        pltpu.make_async_copy(e_ref.at[pl.ds(ebase, EB), pl.ds(0, QW)],
                              ev[k], gsem[k]).wait()
        pltpu.make_async_copy(h_ref.at[sv_all.at[pl.ds(0, EB)]], mv[k],
                              gsem[k]).wait()

    def issue_scatter(b, k):
        pltpu.async_copy(mv[k], agg_sh.at[dv_all.at[pl.ds(b * EB, EB)]],
                         ssem[k], add=True)

    def wait_scatter(k):
        pltpu.make_async_copy(mv[k], agg_sh.at[dv_all.at[pl.ds(0, EB)]],
                              ssem[k]).wait()

    def compute(k):
        mvk, evk = mv[k], ev[k]

        def row(r, carry):
            for j in range(QW // VL):
                sl = pl.ds(j * VL, VL)
                mvk[r, sl] = jnp.maximum(mvk[r, sl] + evk[r, sl], 0.0)
            return carry

        lax.fori_loop(0, EB, row, 0)

    issue_main(0, 0)
    issue_main(1, 1)
    issue_main(2, 2)

    def outer(t, carry):
        for k in range(NSLOT):
            b = NSLOT * t + k
            wait_main(k)
            compute(k)
            issue_scatter(b, k)
            k2 = (k + 3) % NSLOT
            if k < 2:
                @pl.when(t > 0)
                def _():
                    wait_scatter(k2)

                issue_main(b + 3, k2)
            else:
                wait_scatter(k2)

                @pl.when(t < NBLK // NSLOT - 1)
                def _():
                    issue_main(b + 3, k2)
        return carry

    lax.fori_loop(0, NBLK // NSLOT, outer, 0)
    wait_scatter(3)
    wait_scatter(4)
    plsc.subcore_barrier()

    def ochunk(k, carry):
        chunk = s + k * NS

        @pl.when(chunk < NCHUNK)
        def _():
            pltpu.async_copy(agg_sh.at[pl.ds(chunk * EB, EB)],
                             out_ref.at[pl.ds(q * N + chunk * EB, EB)],
                             ssem[0])

        return carry

    def owait(k, carry):
        chunk = s + k * NS

        @pl.when(chunk < NCHUNK)
        def _():
            pltpu.make_async_copy(agg_sh.at[pl.ds(0, EB)],
                                  out_ref.at[pl.ds(q * N, EB)],
                                  ssem[0]).wait()

        return carry

    lax.fori_loop(0, KCH, ochunk, 0)
    lax.fori_loop(0, KCH, owait, 0)
    plsc.subcore_barrier()


def _make_mp():
    return pl.kernel(
        _mp_body,
        out_type=jax.ShapeDtypeStruct((NQ * N, QW), jnp.float32),
        mesh=plsc.VectorSubcoreMesh(core_axis_name="c", subcore_axis_name="s",
                                    num_cores=NC, num_subcores=NS),
        scratch_types=[
            pltpu.VMEM((EPT,), jnp.int32),
            pltpu.VMEM((EPT,), jnp.int32),
            [pltpu.VMEM((EB, QW), jnp.float32) for _ in range(NSLOT)],
            [pltpu.VMEM((EB, QW), jnp.float32) for _ in range(NSLOT)],
            pltpu.VMEM_SHARED((N, QW), jnp.float32),
            [pltpu.SemaphoreType.DMA for _ in range(NSLOT)],
            [pltpu.SemaphoreType.DMA for _ in range(NSLOT)],
        ],
        compiler_params=pltpu.CompilerParams(use_tc_tiling_on_sc=False),
    )


_mp = _make_mp()


def kernel(x, edge_attr, edge_index, batch, Wn, bn, We, be, W1, b1, W2, b2,
           Ws, bs, pw):
    f32 = jnp.float32
    src = edge_index[0]
    dst = edge_index[1]
    qoff = (jnp.arange(NQ, dtype=jnp.int32) * N)[:, None]
    src4 = (src[None, :] + qoff).reshape(NQ * E)
    pad = HP - H
    Wn_p = jnp.pad(Wn, ((0, 0), (0, pad)))
    bn_p = jnp.pad(bn, (0, pad)).reshape(1, HP)
    We_p = jnp.pad(We, ((0, 0), (0, pad)))
    be_p = jnp.pad(be, (0, pad)).reshape(1, HP)
    W1_p = jnp.pad(W1, ((0, 0), (0, pad), (0, pad)))
    b1_p = jnp.pad(b1, ((0, 0), (0, pad)))
    W2_p = jnp.pad(W2, ((0, 0), (0, pad), (0, pad)))
    b2_p = jnp.pad(b2, ((0, 0), (0, pad)))
    Ws_p = jnp.pad(Ws, ((0, pad), (0, 0)))

    h = pl.pallas_call(
        _proj_nodes_body,
        grid=(NROWB,),
        in_specs=[
            pl.BlockSpec((RB, DIN), lambda i: (i, 0)),
            pl.BlockSpec((DIN, HP), lambda i: (0, 0)),
            pl.BlockSpec((1, HP), lambda i: (0, 0)),
        ],
        out_specs=pl.BlockSpec((NQ, RB, QW), lambda i: (0, i, 0)),
        out_shape=jax.ShapeDtypeStruct((NQ, N, QW), f32),
    )(x, Wn_p, bn_p)

    e_flat = pl.pallas_call(
        _proj_edges_body,
        grid=(NQ, EROWB),
        in_specs=[
            pl.BlockSpec((ERB, DEDGE), lambda q, i: (i, 0)),
            pl.BlockSpec((1, DEDGE, QW), lambda q, i: (q, 0, 0)),
            pl.BlockSpec((1, 1, QW), lambda q, i: (q, 0, 0)),
        ],
        out_specs=pl.BlockSpec((ERB, EW), lambda q, i: (q * EROWB + i, 0)),
        out_shape=jax.ShapeDtypeStruct((NQ * E, EW), f32),
    )(edge_attr,
      We_p.reshape(DEDGE, NQ, QW).transpose(1, 0, 2),
      be_p.reshape(1, NQ, QW).transpose(1, 0, 2))

    for i in range(DEPTH):
        h_flat = h.reshape(NQ * N, QW)
        agg = _mp(h_flat, e_flat, src4, dst)
        h = pl.pallas_call(
            functools.partial(_mlp_body, last=(i == DEPTH - 1)),
            grid=(NROWB,),
            in_specs=[
                pl.BlockSpec((NQ, RB, QW), lambda i: (0, i, 0)),
                pl.BlockSpec((NQ, RB, QW), lambda i: (0, i, 0)),
                pl.BlockSpec((HP, HP), lambda i: (0, 0)),
                pl.BlockSpec((1, HP), lambda i: (0, 0)),
                pl.BlockSpec((HP, HP), lambda i: (0, 0)),
                pl.BlockSpec((1, HP), lambda i: (0, 0)),
            ],
            out_specs=pl.BlockSpec((NQ, RB, QW), lambda i: (0, i, 0)),
            out_shape=jax.ShapeDtypeStruct((NQ, N, QW), f32),
        )(h, agg.reshape(NQ, N, QW),
          W1_p[i], b1_p[i].reshape(1, HP), W2_p[i], b2_p[i].reshape(1, HP))

    out = pl.pallas_call(
        _readout_body,
        grid=(NROWB,),
        in_specs=[
            pl.BlockSpec((NQ, RB, QW), lambda i: (0, i, 0)),
            pl.BlockSpec((1, 1, RB), lambda i: (i, 0, 0)),
            pl.BlockSpec((HP, R), lambda i: (0, 0)),
            pl.BlockSpec((1, R), lambda i: (0, 0)),
            pl.BlockSpec((1, 1), lambda i: (0, 0)),
        ],
        out_specs=pl.BlockSpec((G, R), lambda i: (0, 0)),
        out_shape=jax.ShapeDtypeStruct((G, R), f32),
        scratch_shapes=[pltpu.VMEM((G, HP), f32)],
    )(h, batch.reshape(NROWB, 1, RB), Ws_p, bs.reshape(1, R),
      pw.reshape(1, 1))
    return out
